# Initial kernel scaffold; baseline (speedup 1.0000x reference)
import jax
import jax.numpy as jnp
from jax.experimental import pallas as pl


def kernel(x_low, edge_index_low, x_high, z_std_high, edge_index_low2high, edge_index_high, params):
    raise NotImplementedError("write your pallas kernel here")


# speedup vs baseline: 52.4151x; 52.4151x over previous
import functools

import jax
import jax.numpy as jnp
from jax import lax
from jax.experimental import pallas as pl
from jax.experimental.pallas import tpu as pltpu
from jax.experimental.pallas import tpu_sc as plsc

F32 = jnp.float32
I32 = jnp.int32
KE = 128
NSUB = 16
EPS_DEN = 1e-16
BN_EPS = 1e-5


def _mesh():
    return plsc.VectorSubcoreMesh(core_axis_name="c", subcore_axis_name="s",
                                  num_cores=2, num_subcores=NSUB)


def _sc_count(dst_idx, n):
    e = dst_idx.shape[0]
    assert e % (2 * KE) == 0 and n % NSUB == 0
    ec = e // 2
    nchunk = ec // KE

    ZK = 2048
    nz = n // ZK
    tz = n % ZK
    na = ((n + ZK - 1) // ZK) * ZK

    @functools.partial(
        pl.kernel,
        out_type=jax.ShapeDtypeStruct((2 * na,), F32),
        name=f"sc_count_{e}_{n}",
        mesh=_mesh(),
        compiler_params=pltpu.CompilerParams(needs_layout_passes=False,
                                             use_tc_tiling_on_sc=False),
        scratch_types=[
            pltpu.VMEM_SHARED((n,), F32),
            pltpu.VMEM((KE,), I32),
            pltpu.VMEM((KE,), F32),
            pltpu.VMEM((ZK,), F32),
        ],
    )
    def k(dst_hbm, out_hbm, acc, idx_v, ones_v, zb_v):
        c = lax.axis_index("c")
        s = lax.axis_index("s")

        def fill1(i, carry):
            ones_v[pl.ds(i * 16, 16)] = jnp.ones((16,), F32)
            return carry

        lax.fori_loop(0, KE // 16, fill1, 0)

        @pl.when(s == 0)
        def _():
            def fz(i, carry):
                zb_v[pl.ds(i * 16, 16)] = jnp.zeros((16,), F32)
                return carry

            lax.fori_loop(0, ZK // 16, fz, 0)

            def z(i, carry):
                pltpu.sync_copy(zb_v, acc.at[pl.ds(i * ZK, ZK)])
                return carry

            lax.fori_loop(0, nz, z, 0)
            if tz:
                pltpu.sync_copy(zb_v.at[pl.ds(0, tz)],
                                acc.at[pl.ds(nz * ZK, tz)])

        plsc.subcore_barrier()
        nme = (nchunk - s + NSUB - 1) // NSUB

        def body(j, carry):
            chunk = s + j * NSUB
            base = c * ec + chunk * KE
            pltpu.sync_copy(dst_hbm.at[pl.ds(base, KE)], idx_v)
            pltpu.sync_copy(ones_v, acc.at[idx_v], add=True)
            return carry

        lax.fori_loop(0, nme, body, 0)
        plsc.subcore_barrier()

        @pl.when(s == 0)
        def _():
            def w(i, carry):
                pltpu.sync_copy(acc.at[pl.ds(i * ZK, ZK)], zb_v)
                pltpu.sync_copy(zb_v, out_hbm.at[pl.ds(c * na + i * ZK, ZK)])
                return carry

            lax.fori_loop(0, nz, w, 0)
            if tz:
                pltpu.sync_copy(acc.at[pl.ds(nz * ZK, tz)],
                                zb_v.at[pl.ds(0, tz)])
                pltpu.sync_copy(zb_v.at[pl.ds(0, tz)],
                                out_hbm.at[pl.ds(c * na + nz * ZK, tz)])

    out = k(dst_idx)
    return out.reshape(2, na)[:, :n]


def _sc_segsum(table, src_idx, dst_idx, n, C):
    e = src_idx.shape[0]
    assert e % (2 * KE) == 0 and n % NSUB == 0
    ec = e // 2
    nchunk = ec // KE
    nt = n // KE
    tl = n % KE
    assert tl % 8 == 0

    @functools.partial(
        pl.kernel,
        out_type=jax.ShapeDtypeStruct((2 * C * n, 128), F32),
        name=f"sc_segsum_{e}_{n}_{C}",
        mesh=_mesh(),
        compiler_params=pltpu.CompilerParams(needs_layout_passes=False,
                                             use_tc_tiling_on_sc=False),
        scratch_types=[
            pltpu.VMEM_SHARED((n, 128), F32),
            pltpu.VMEM((KE,), I32),
            pltpu.VMEM((KE,), I32),
            pltpu.VMEM((KE,), I32),
            pltpu.VMEM((KE, 128), F32),
            pltpu.SemaphoreType.DMA,
        ],
    )
    def k(tab_hbm, src_hbm, dst_hbm, out_hbm,
          acc, src_v, dst_v, idx_v, rows_v, gsem):
        c = lax.axis_index("c")
        s = lax.axis_index("s")
        nq = (nt - s + NSUB - 1) // NSUB
        for ci in range(C):
            def fz(i, carry):
                for jj in range(8):
                    rows_v[i, pl.ds(jj * 16, 16)] = jnp.zeros((16,), F32)
                return carry

            lax.fori_loop(0, KE, fz, 0)

            def zcp(i, carry):
                q = s + i * NSUB
                pltpu.sync_copy(rows_v, acc.at[pl.ds(q * KE, KE)])
                return carry

            lax.fori_loop(0, nq, zcp, 0)
            if tl:
                @pl.when(s == 0)
                def _():
                    pltpu.sync_copy(rows_v.at[pl.ds(0, tl)],
                                    acc.at[pl.ds(nt * KE, tl)])
            plsc.subcore_barrier()
            nme = (nchunk - s + NSUB - 1) // NSUB

            def body(j, carry):
                chunk = s + j * NSUB
                base = c * ec + chunk * KE
                pltpu.sync_copy(src_hbm.at[pl.ds(base, KE)], src_v)
                pltpu.sync_copy(dst_hbm.at[pl.ds(base, KE)], dst_v)

                def mkidx(q, carry2):
                    sl = pl.ds(q * 16, 16)
                    idx_v[sl] = src_v[sl] * C + ci
                    return carry2

                lax.fori_loop(0, KE // 16, mkidx, 0)
                pltpu.async_copy(tab_hbm.at[idx_v], rows_v, gsem).wait()
                pltpu.sync_copy(rows_v, acc.at[dst_v], add=True)
                return carry

            lax.fori_loop(0, nme, body, 0)
            plsc.subcore_barrier()
            obase = (c * C + ci) * n

            def ocp(i, carry):
                q = s + i * NSUB
                pltpu.sync_copy(acc.at[pl.ds(q * KE, KE)], rows_v)
                pltpu.sync_copy(rows_v, out_hbm.at[pl.ds(obase + q * KE, KE)])
                return carry

            lax.fori_loop(0, nq, ocp, 0)
            if tl:
                @pl.when(s == 0)
                def _():
                    pltpu.sync_copy(acc.at[pl.ds(nt * KE, tl)],
                                    rows_v.at[pl.ds(0, tl)])
                    pltpu.sync_copy(rows_v.at[pl.ds(0, tl)],
                                    out_hbm.at[pl.ds(obase + nt * KE, tl)])
            plsc.subcore_barrier()

    return k(table, src_idx, dst_idx)


def _sc_gat(hsT, hdT, src_idx, dst_idx, att_flat, n_dst, stride, head_base,
            split_edges):
    e = src_idx.shape[0]
    assert e % (2 * KE) == 0 and n_dst % NSUB == 0
    if split_edges:
        ec = e // 2
    else:
        ec = e
    nchunk = ec // KE
    nt = n_dst // KE
    tl = n_dst % KE
    ZK = 2048
    nz = n_dst // ZK
    tz = n_dst % ZK
    na = ((n_dst + ZK - 1) // ZK) * ZK
    assert tl % 8 == 0
    hsz = att_flat.shape[0]

    @functools.partial(
        pl.kernel,
        out_type=(jax.ShapeDtypeStruct((2 * n_dst, 16), F32),
                  jax.ShapeDtypeStruct((2 * na,), F32)),
        name=f"sc_gat_{e}_{n_dst}_{stride}_{head_base}_{int(split_edges)}",
        mesh=_mesh(),
        compiler_params=pltpu.CompilerParams(needs_layout_passes=False,
                                             use_tc_tiling_on_sc=False),
        scratch_types=[
            pltpu.VMEM_SHARED((n_dst, 16), F32),
            pltpu.VMEM_SHARED((n_dst,), F32),
            pltpu.VMEM((KE,), I32),
            pltpu.VMEM((KE,), I32),
            pltpu.VMEM((KE,), I32),
            pltpu.VMEM((KE,), I32),
            pltpu.VMEM((KE, 16), F32),
            pltpu.VMEM((KE, 16), F32),
            pltpu.VMEM((KE, 16), F32),
            pltpu.VMEM((KE,), F32),
            pltpu.VMEM((hsz,), F32),
            pltpu.VMEM((ZK,), F32),
            pltpu.SemaphoreType.DMA,
            pltpu.SemaphoreType.DMA,
        ],
    )
    def k(hs_hbm, hd_hbm, src_hbm, dst_hbm, att_hbm,
          num_out, den_out,
          numacc, denacc, src_v, dst_v, idxs_v, idxd_v,
          xj_v, xi_v, wx_v, w_v, att_v, zb_v, sem_j, sem_i):
        c = lax.axis_index("c")
        s = lax.axis_index("s")
        if split_edges:
            h = head_base + c * 0
            ebase = c * ec
        else:
            h = head_base + c
            ebase = c * 0
        pltpu.sync_copy(att_hbm, att_v)
        att_vec = att_v[pl.ds(h * 16, 16)]
        att_sc = [att_vec[i] for i in range(16)]

        def fill0(i, carry):
            wx_v[i, :] = jnp.zeros((16,), F32)
            return carry

        lax.fori_loop(0, KE, fill0, 0)
        nq = (nt - s + NSUB - 1) // NSUB

        def zcp(i, carry):
            q = s + i * NSUB
            pltpu.sync_copy(wx_v, numacc.at[pl.ds(q * KE, KE)])
            return carry

        lax.fori_loop(0, nq, zcp, 0)
        if tl:
            @pl.when(s == 1)
            def _():
                pltpu.sync_copy(wx_v.at[pl.ds(0, tl)],
                                numacc.at[pl.ds(nt * KE, tl)])

        @pl.when(s == 0)
        def _():
            def fz(i, carry):
                zb_v[pl.ds(i * 16, 16)] = jnp.zeros((16,), F32)
                return carry

            lax.fori_loop(0, ZK // 16, fz, 0)

            def z(i, carry):
                pltpu.sync_copy(zb_v, denacc.at[pl.ds(i * ZK, ZK)])
                return carry

            lax.fori_loop(0, nz, z, 0)
            if tz:
                pltpu.sync_copy(zb_v.at[pl.ds(0, tz)],
                                denacc.at[pl.ds(nz * ZK, tz)])

        plsc.subcore_barrier()
        nme = (nchunk - s + NSUB - 1) // NSUB

        def body(j, carry):
            chunk = s + j * NSUB
            base = ebase + chunk * KE
            pltpu.sync_copy(src_hbm.at[pl.ds(base, KE)], src_v)
            pltpu.sync_copy(dst_hbm.at[pl.ds(base, KE)], dst_v)

            def mkidx(q, carry2):
                sl = pl.ds(q * 16, 16)
                idxs_v[sl] = src_v[sl] * stride + h
                idxd_v[sl] = dst_v[sl] * stride + h
                return carry2

            lax.fori_loop(0, KE // 16, mkidx, 0)
            cpj = pltpu.async_copy(hs_hbm.at[idxs_v], xj_v, sem_j)
            cpi = pltpu.async_copy(hd_hbm.at[idxd_v], xi_v, sem_i)
            cpj.wait()
            cpi.wait()
            for g in range(KE // 16):
                rowi = lax.iota(I32, 16) + (g * 16)
                acc16 = jnp.zeros((16,), F32)
                for ch in range(16):
                    colv = jnp.full((16,), ch, I32)
                    xjc = plsc.load_gather(xj_v, [rowi, colv])
                    xic = plsc.load_gather(xi_v, [rowi, colv])
                    sm = xic + xjc
                    lk = jnp.maximum(sm, 0.2 * sm)
                    acc16 = acc16 + lk * att_sc[ch]
                wv = jnp.exp(acc16)
                w_v[pl.ds(g * 16, 16)] = wv
                for ch in range(16):
                    colv = jnp.full((16,), ch, I32)
                    xjc = plsc.load_gather(xj_v, [rowi, colv])
                    plsc.store_scatter(wx_v, [rowi, colv], wv * xjc)
            pltpu.sync_copy(wx_v, numacc.at[dst_v], add=True)
            pltpu.sync_copy(w_v, denacc.at[dst_v], add=True)
            return carry

        lax.fori_loop(0, nme, body, 0)
        plsc.subcore_barrier()
        obase = c * n_dst

        def ocp(i, carry):
            q = s + i * NSUB
            pltpu.sync_copy(numacc.at[pl.ds(q * KE, KE)], wx_v)
            pltpu.sync_copy(wx_v, num_out.at[pl.ds(obase + q * KE, KE)])
            return carry

        lax.fori_loop(0, nq, ocp, 0)
        if tl:
            @pl.when(s == 1)
            def _():
                pltpu.sync_copy(numacc.at[pl.ds(nt * KE, tl)],
                                wx_v.at[pl.ds(0, tl)])
                pltpu.sync_copy(wx_v.at[pl.ds(0, tl)],
                                num_out.at[pl.ds(obase + nt * KE, tl)])

        @pl.when(s == 0)
        def _():
            dbase = c * na

            def w(i, carry):
                pltpu.sync_copy(denacc.at[pl.ds(i * ZK, ZK)], zb_v)
                pltpu.sync_copy(
                    zb_v, den_out.at[pl.ds(dbase + i * ZK, ZK)])
                return carry

            lax.fori_loop(0, nz, w, 0)
            if tz:
                pltpu.sync_copy(denacc.at[pl.ds(nz * ZK, tz)],
                                zb_v.at[pl.ds(0, tz)])
                pltpu.sync_copy(zb_v.at[pl.ds(0, tz)],
                                den_out.at[pl.ds(dbase + nz * ZK, tz)])

    num, den = k(hsT, hdT, src_idx, dst_idx, att_flat)
    return num.reshape(2, n_dst, 16), den.reshape(2, na)[:, :n_dst]


def _rows(a, blk):
    nd = a.ndim
    return pl.BlockSpec((blk,) + a.shape[1:],
                        lambda i, nd=nd: (i,) + (0,) * (nd - 1))


def _rows1(a, blk):
    nd = a.ndim
    return pl.BlockSpec((a.shape[0], blk) + a.shape[2:],
                        lambda i, nd=nd: (0, i) + (0,) * (nd - 2))


def _whole(a):
    nd = a.ndim
    return pl.BlockSpec(a.shape, lambda i, nd=nd: (0,) * nd)


def _stats_spec(f):
    return pl.BlockSpec((1, 2, f), lambda i: (i, 0, 0))


def _bn_from_stats(x, st, g, b, n):
    s1 = jnp.sum(st[:, 0, :], axis=0, keepdims=True)
    s2 = jnp.sum(st[:, 1, :], axis=0, keepdims=True)
    mu = s1 / n
    var = s2 / n - mu * mu
    return (x - mu) * lax.rsqrt(var + BN_EPS) * g + b


def _emit_stats(x):
    s1 = jnp.sum(x, axis=0, keepdims=True)
    s2 = jnp.sum(x * x, axis=0, keepdims=True)
    return jnp.concatenate([s1, s2], axis=0)[None]


def _leaky(x):
    return jnp.maximum(x, 0.2 * x)


def _t_low_prep(xt, degT, blk):
    n = xt.shape[0]

    def body(x_ref, d_ref, xp_ref, dinv_ref):
        deg = d_ref[:, 0:1] + d_ref[:, 1:2] + 1.0
        dinv = lax.rsqrt(deg)
        dinv_ref[...] = dinv
        xp_ref[...] = x_ref[...] * dinv

    return pl.pallas_call(
        body,
        grid=(n // blk,),
        in_specs=[_rows(xt, blk), _rows(degT, blk)],
        out_specs=[_rows(xt, blk), pl.BlockSpec((blk, 1), lambda i: (i, 0))],
        out_shape=[jax.ShapeDtypeStruct((n, 640), F32),
                   jax.ShapeDtypeStruct((n, 1), F32)],
    )(xt, degT)


def _t_temporal(Sp, Xp, dinv, p, blk):
    n = Xp.shape[0]
    czW = p['cz_W']
    chW = p['ch_W']
    lzW = p['lz_W']
    lhW = p['lh_W']
    czb = p['cz_b'].reshape(1, 25)
    chb = p['ch_b'].reshape(1, 25)
    lzb = p['lz_b'].reshape(1, 25)
    lhb = p['lh_b'].reshape(1, 25)
    attp = p['att_p'].reshape(1, 25)
    dlW = p['dl_W']

    def body(S_ref, xp_ref, dinv_ref, czW_r, chW_r, lzW_r, lhW_r,
             czb_r, chb_r, lzb_r, lhb_r, attp_r, dlW_r, hs_ref):
        parts = [S_ref[ci] + S_ref[5 + ci] for ci in range(5)]
        scat = jnp.concatenate(parts, axis=1)
        ax = (scat + xp_ref[...]) * dinv_ref[...]
        lz1 = lzW_r[...][:25, :]
        lh1 = lhW_r[...][:25, :]
        Wz = jnp.dot(czW_r[...], lz1, preferred_element_type=F32)
        Wh = jnp.dot(chW_r[...], lh1, preferred_element_type=F32)
        bz = jnp.dot(czb_r[...], lz1, preferred_element_type=F32) + lzb_r[...]
        bh = jnp.dot(chb_r[...], lh1, preferred_element_type=F32) + lhb_r[...]
        probs = jax.nn.softmax(attp_r[...], axis=-1)
        acc = jnp.zeros((ax.shape[0], 25), F32)
        for t in range(25):
            G = ax[:, 25 * t:25 * t + 25]
            Z = jax.nn.sigmoid(jnp.dot(G, Wz, preferred_element_type=F32) + bz)
            Ht = jnp.tanh(jnp.dot(G, Wh, preferred_element_type=F32) + bh)
            acc = acc + probs[0, t] * (1.0 - Z) * Ht
        hs_ref[...] = jnp.dot(acc, dlW_r[...], preferred_element_type=F32)

    small = [czW, chW, lzW, lhW, czb, chb, lzb, lhb, attp, dlW]
    return pl.pallas_call(
        body,
        grid=(n // blk,),
        in_specs=[_rows1(Sp, blk), _rows(Xp, blk),
                  pl.BlockSpec((blk, 1), lambda i: (i, 0))]
                 + [_whole(a) for a in small],
        out_specs=_rows(jnp.empty((n, 64)), blk),
        out_shape=jax.ShapeDtypeStruct((n, 64), F32),
    )(Sp, Xp, dinv, *small)


def _t_hd(x_high, drW, blk):
    n = x_high.shape[0]
    drW2 = drW.reshape(1, 64)

    def body(x_ref, w_ref, o_ref):
        o_ref[...] = x_ref[...] * w_ref[...]

    return pl.pallas_call(
        body,
        grid=(n // blk,),
        in_specs=[pl.BlockSpec((blk, 1), lambda i: (i, 0)), _whole(drW2)],
        out_specs=_rows(jnp.empty((n, 64)), blk),
        out_shape=jax.ShapeDtypeStruct((n, 64), F32),
    )(x_high, drW2)


def _t_ds_post(num4, den4, cnt2, z_std, db, blk):
    n = z_std.shape[0]
    nb = n // blk

    def body(num_ref, den_ref, cnt_ref, z_ref, db_ref, x0_ref, st_ref):
        cnt = jnp.maximum(cnt_ref[:, 0:1] + cnt_ref[:, 1:2], 1.0)
        acc = jnp.zeros((cnt.shape[0], 16), F32)
        for h in range(4):
            acc = acc + num_ref[h] / jnp.maximum(den_ref[:, h:h + 1], EPS_DEN)
        e12 = acc * 0.25 / cnt + db_ref[...]
        x0 = jnp.concatenate([z_ref[...], e12], axis=1)
        x0_ref[...] = x0
        st_ref[...] = _emit_stats(x0)

    return pl.pallas_call(
        body,
        grid=(nb,),
        in_specs=[_rows1(num4, blk), _rows(den4, blk), _rows(cnt2, blk),
                  pl.BlockSpec((blk, 1), lambda i: (i, 0)), _whole(db)],
        out_specs=[_rows(jnp.empty((n, 17)), blk), _stats_spec(17)],
        out_shape=[jax.ShapeDtypeStruct((n, 17), F32),
                   jax.ShapeDtypeStruct((nb, 2, 17), F32)],
    )(num4, den4, cnt2, z_std, db)


def _t_pre(y, st, g, b, Wl, Wr, use_relu, blk):
    n = y.shape[0]
    fo = Wl.shape[1]

    def body(y_ref, st_ref, g_ref, b_ref, wl_ref, wr_ref, hs_ref, hd_ref):
        t = _bn_from_stats(y_ref[...], st_ref[...], g_ref[...], b_ref[...], n)
        if use_relu:
            t = jnp.maximum(t, 0.0)
        hs_ref[...] = jnp.dot(t, wl_ref[...], preferred_element_type=F32)
        hd_ref[...] = jnp.dot(t, wr_ref[...], preferred_element_type=F32)

    return pl.pallas_call(
        body,
        grid=(n // blk,),
        in_specs=[_rows(y, blk), _whole(st), _whole(g), _whole(b),
                  _whole(Wl), _whole(Wr)],
        out_specs=[_rows(jnp.empty((n, fo)), blk)] * 2,
        out_shape=[jax.ShapeDtypeStruct((n, fo), F32)] * 2,
    )(y, st, g, b, Wl, Wr)


def _t_post(num2, den2, cnt2, hs, hd, att, gb, blk):
    n = hs.shape[0]
    nb = n // blk

    def body(num_ref, den_ref, cnt_ref, hs_ref, hd_ref, att_ref, gb_ref,
             y_ref, st_ref):
        cntt = cnt_ref[:, 0:1] + cnt_ref[:, 1:2] + 1.0
        outs = []
        for h in range(2):
            hsh = hs_ref[:, 16 * h:16 * h + 16]
            hdh = hd_ref[:, 16 * h:16 * h + 16]
            lk = _leaky(hsh + hdh)
            wl = jnp.exp(jnp.sum(lk * att_ref[h:h + 1, :], axis=1,
                                 keepdims=True))
            nh = num_ref[h] + wl * hsh
            dh = den_ref[:, h:h + 1] + wl
            outs.append(nh / jnp.maximum(dh, EPS_DEN) / cntt)
        y = jnp.concatenate(outs, axis=1) + gb_ref[...]
        y_ref[...] = y
        st_ref[...] = _emit_stats(y)

    return pl.pallas_call(
        body,
        grid=(nb,),
        in_specs=[_rows1(num2, blk), _rows(den2, blk), _rows(cnt2, blk),
                  _rows(hs, blk), _rows(hd, blk), _whole(att), _whole(gb)],
        out_specs=[_rows(jnp.empty((n, 32)), blk), _stats_spec(32)],
        out_shape=[jax.ShapeDtypeStruct((n, 32), F32),
                   jax.ShapeDtypeStruct((nb, 2, 32), F32)],
    )(num2, den2, cnt2, hs, hd, att, gb)


def _t_post5(num2, den2, cnt2, hs, hd, att, gb, p1W, p1b, blk):
    n = hs.shape[0]
    nb = n // blk

    def body(num_ref, den_ref, cnt_ref, hs_ref, hd_ref, att_ref, gb_ref,
             w_ref, b_ref, h1_ref, st_ref):
        cntt = cnt_ref[:, 0:1] + cnt_ref[:, 1:2] + 1.0
        lk = _leaky(hs_ref[...] + hd_ref[...])
        wl = jnp.exp(jnp.sum(lk * att_ref[...], axis=1, keepdims=True))
        nh = num_ref[0] + num_ref[1] + wl * hs_ref[...]
        dh = den_ref[:, 0:1] + den_ref[:, 1:2] + wl
        y5 = nh / jnp.maximum(dh, EPS_DEN) / cntt + gb_ref[...]
        x = jnp.maximum(y5, 0.0)
        h1 = jnp.maximum(jnp.dot(x, w_ref[...], preferred_element_type=F32)
                         + b_ref[...], 0.0)
        h1_ref[...] = h1
        st_ref[...] = _emit_stats(h1)

    return pl.pallas_call(
        body,
        grid=(nb,),
        in_specs=[_rows1(num2, blk), _rows(den2, blk), _rows(cnt2, blk),
                  _rows(hs, blk), _rows(hd, blk), _whole(att), _whole(gb),
                  _whole(p1W), _whole(p1b)],
        out_specs=[_rows(jnp.empty((n, 16)), blk), _stats_spec(16)],
        out_shape=[jax.ShapeDtypeStruct((n, 16), F32),
                   jax.ShapeDtypeStruct((nb, 2, 16), F32)],
    )(num2, den2, cnt2, hs, hd, att, gb, p1W, p1b)


def _t_mlp_mid(h1, st, g, b, W, bb, blk):
    n = h1.shape[0]
    nb = n // blk
    fo = W.shape[1]

    def body(h_ref, st_ref, g_ref, b_ref, w_ref, bb_ref, o_ref, so_ref):
        t = _bn_from_stats(h_ref[...], st_ref[...], g_ref[...], b_ref[...], n)
        o = jnp.maximum(jnp.dot(t, w_ref[...], preferred_element_type=F32)
                        + bb_ref[...], 0.0)
        o_ref[...] = o
        so_ref[...] = _emit_stats(o)

    return pl.pallas_call(
        body,
        grid=(nb,),
        in_specs=[_rows(h1, blk), _whole(st), _whole(g), _whole(b),
                  _whole(W), _whole(bb)],
        out_specs=[_rows(jnp.empty((n, fo)), blk), _stats_spec(fo)],
        out_shape=[jax.ShapeDtypeStruct((n, fo), F32),
                   jax.ShapeDtypeStruct((nb, 2, fo), F32)],
    )(h1, st, g, b, W, bb)


def _t_mlp_out(h2, st, g, b, W, bb, blk):
    n = h2.shape[0]

    def body(h_ref, st_ref, g_ref, b_ref, w_ref, bb_ref, o_ref):
        t = _bn_from_stats(h_ref[...], st_ref[...], g_ref[...], b_ref[...], n)
        o_ref[...] = jnp.dot(t, w_ref[...], preferred_element_type=F32) \
            + bb_ref[...]

    return pl.pallas_call(
        body,
        grid=(n // blk,),
        in_specs=[_rows(h2, blk), _whole(st), _whole(g), _whole(b),
                  _whole(W), _whole(bb)],
        out_specs=pl.BlockSpec((blk, 1), lambda i: (i, 0)),
        out_shape=jax.ShapeDtypeStruct((n, 1), F32),
    )(h2, st, g, b, W, bb)


def kernel(x_low, edge_index_low, x_high, z_std_high, edge_index_low2high,
           edge_index_high, params):
    p = params
    nl = x_low.shape[0]
    nh = x_high.shape[0]
    blk_l = 2000
    blk_h = 2000

    src_l = edge_index_low[0]
    dst_l = edge_index_low[1]
    src_d = edge_index_low2high[0]
    dst_d = edge_index_low2high[1]
    src_h = edge_index_high[0]
    dst_h = edge_index_high[1]

    degp = _sc_count(dst_l, nl)
    xt = jnp.pad(jnp.transpose(x_low, (0, 2, 1)).reshape(nl, 625),
                 ((0, 0), (0, 15)))
    Xp, dinv = _t_low_prep(xt, jnp.transpose(degp), blk_l)
    tab = Xp.reshape(nl * 5, 128)
    Sp = _sc_segsum(tab, src_l, dst_l, nl, 5).reshape(10, nl, 128)
    hs_d = _t_temporal(Sp, Xp, dinv, p, blk_l)

    hd_d = _t_hd(x_high, p['dr_W'], blk_h)
    cnt_d = _sc_count(dst_d, nh)
    hsT = hs_d.reshape(nl * 4, 16)
    hdT = hd_d.reshape(nh * 4, 16)
    att_d = p['d_att'].reshape(64)
    n0, d0 = _sc_gat(hsT, hdT, src_d, dst_d, att_d, nh, 4, 0, False)
    n1, d1 = _sc_gat(hsT, hdT, src_d, dst_d, att_d, nh, 4, 2, False)
    num4 = jnp.concatenate([n0, n1], axis=0)
    den4 = jnp.transpose(jnp.concatenate([d0, d1], axis=0))
    x0, st0 = _t_ds_post(num4, den4, jnp.transpose(cnt_d), z_std_high,
                         p['d_b'].reshape(1, 16), blk_h)

    cnt_h = jnp.transpose(_sc_count(dst_h, nh))
    hs, hd = _t_pre(x0, st0, p['bn0_g'].reshape(1, 17),
                    p['bn0_b'].reshape(1, 17), p['g1_l'], p['g1_r'],
                    False, blk_h)
    for l in range(1, 5):
        att = p[f'g{l}_att']
        num2, den2 = _sc_gat(hs.reshape(nh * 2, 16), hd.reshape(nh * 2, 16),
                             src_h, dst_h, att.reshape(32), nh, 2, 0, False)
        y, st = _t_post(num2, jnp.transpose(den2), cnt_h, hs, hd, att,
                        p[f'g{l}_b'].reshape(1, 32), blk_h)
        hs, hd = _t_pre(y, st, p[f'bn{l}_g'].reshape(1, 32),
                        p[f'bn{l}_b'].reshape(1, 32),
                        p[f'g{l + 1}_l'], p[f'g{l + 1}_r'], True, blk_h)
    num5, den5 = _sc_gat(hs, hd, src_h, dst_h, p['g5_att'].reshape(16),
                         nh, 1, 0, True)
    h1, st1 = _t_post5(num5, jnp.transpose(den5), cnt_h, hs, hd,
                       p['g5_att'], p['g5_b'].reshape(1, 16),
                       p['p1_W'], p['p1_b'].reshape(1, 16), blk_h)
    h2, st2 = _t_mlp_mid(h1, st1, p['pbn1_g'].reshape(1, 16),
                         p['pbn1_b'].reshape(1, 16), p['p2_W'],
                         p['p2_b'].reshape(1, 8), blk_h)
    return _t_mlp_out(h2, st2, p['pbn2_g'].reshape(1, 8),
                      p['pbn2_b'].reshape(1, 8), p['p3_W'],
                      p['p3_b'].reshape(1, 1), blk_h)
